# baseline (speedup 1.0000x reference)
import functools

import jax
import jax.numpy as jnp
from jax.experimental import pallas as pl
from jax.experimental.pallas import tpu as pltpu

NUM_GROUP = 256
GROUP_SIZE = 32
_N_SIDE = 128
_N_PTS = 16384
_B = 8
_NT = 4096
_N_TILES = _N_PTS // _NT
_N_CHUNK = 512
_CPT = _NT // 32


def _oems_pairs(n):
    pairs = []
    p = 1
    while p < n:
        k = p
        while k >= 1:
            for j in range(k % p, n - k, 2 * k):
                for i in range(0, k):
                    if (i + j) // (2 * p) == (i + j + k) // (2 * p):
                        pairs.append((i + j, i + j + k))
            k //= 2
        p *= 2
    return pairs


def _bitonic_merge_pairs(n):
    pairs = []
    d = n // 2
    while d >= 1:
        for i in range(n):
            if (i & d) == 0 and i + d < n:
                pairs.append((i, i + d))
        d //= 2
    return pairs


_SORT32 = _oems_pairs(32)
_MERGE32 = _bitonic_merge_pairs(32)


def _cmp_lex(keys, pay, i, j):
    ka, kb = keys[i], keys[j]
    pa, pb = pay[i], pay[j]
    pred = (ka < kb) | ((ka == kb) & (pa < pb))
    keys[i] = jnp.where(pred, ka, kb)
    keys[j] = jnp.where(pred, kb, ka)
    pay[i] = jnp.where(pred, pa, pb)
    pay[j] = jnp.where(pred, pb, pa)


def _top32_tournament(keys, pay):
    for i, j in _SORT32:
        _cmp_lex(keys, pay, i, j)
    L = keys[0].shape[0]
    while L > 1:
        h = L // 2
        ka = [keys[w][:h] for w in range(32)]
        kb = [keys[31 - w][h:] for w in range(32)]
        pa = [pay[w][:h] for w in range(32)]
        pb = [pay[31 - w][h:] for w in range(32)]
        for w in range(32):
            pred = (ka[w] < kb[w]) | ((ka[w] == kb[w]) & (pa[w] < pb[w]))
            keys[w] = jnp.where(pred, ka[w], kb[w])
            pay[w] = jnp.where(pred, pa[w], pb[w])
        for i, j in _MERGE32:
            _cmp_lex(keys, pay, i, j)
        L = h
    return keys, pay


def _fps_kernel(x_ref, y_ref, z_ref, cx_ref, cy_ref, cz_ref):
    X = x_ref[0]
    Y = y_ref[0]
    Z = z_ref[0]
    rows = jax.lax.broadcasted_iota(jnp.int32, (_N_SIDE, _N_SIDE), 0)
    cols = jax.lax.broadcasted_iota(jnp.int32, (_N_SIDE, _N_SIDE), 1)
    iota_flat = rows * _N_SIDE + cols

    def body(i, carry):
        last, dists = carry
        mask = iota_flat == last
        px = jnp.sum(jnp.where(mask, X, 0.0))
        py = jnp.sum(jnp.where(mask, Y, 0.0))
        pz = jnp.sum(jnp.where(mask, Z, 0.0))
        cx_ref[0, pl.ds(i, 1), :] = jnp.full((1, _N_SIDE), px, dtype=jnp.float32)
        cy_ref[0, pl.ds(i, 1), :] = jnp.full((1, _N_SIDE), py, dtype=jnp.float32)
        cz_ref[0, pl.ds(i, 1), :] = jnp.full((1, _N_SIDE), pz, dtype=jnp.float32)
        dx = X - px
        dy = Y - py
        dz = Z - pz
        d = dx * dx + dy * dy + dz * dz
        dists = jnp.minimum(dists, d)
        m = jnp.max(dists)
        amask = dists == m
        nxt = jnp.min(jnp.where(amask, iota_flat, jnp.int32(2**30)))
        return (nxt, dists)

    dists0 = jnp.full((_N_SIDE, _N_SIDE), 1e10, dtype=jnp.float32)
    jax.lax.fori_loop(0, NUM_GROUP, body, (jnp.int32(0), dists0))


def _fps_pallas(x, y, z):
    in_spec = pl.BlockSpec((1, _N_SIDE, _N_SIDE), lambda b: (b, 0, 0))
    out_spec = pl.BlockSpec((1, NUM_GROUP, _N_SIDE), lambda b: (b, 0, 0))
    out_shape = jax.ShapeDtypeStruct((_B, NUM_GROUP, _N_SIDE), jnp.float32)
    return pl.pallas_call(
        _fps_kernel,
        grid=(_B,),
        in_specs=[in_spec, in_spec, in_spec],
        out_specs=[out_spec, out_spec, out_spec],
        out_shape=[out_shape, out_shape, out_shape],
    )(x, y, z)


def _k1_kernel(p8_ref, c8_ref, d_ref, cid_ref, mt_ref):
    nt = pl.program_id(1)
    P = p8_ref[0]
    C = c8_ref[0]
    qq = (C[:, 0:1] * C[:, 0:1] + C[:, 1:2] * C[:, 1:2]) + C[:, 2:3] * C[:, 2:3]
    rr = (P[0:1, :] * P[0:1, :] + P[1:2, :] * P[1:2, :]) + P[2:3, :] * P[2:3, :]
    dot = jax.lax.dot_general(C, P, (((1,), (0,)), ((), ())),
                              preferred_element_type=jnp.float32)
    Dt = (qq + rr) - 2.0 * dot
    d_ref[0] = Dt

    m = jnp.min(Dt.reshape(NUM_GROUP, _CPT, 32), axis=2)
    mt_ref[:, pl.ds(nt * _CPT, _CPT)] = m

    @pl.when(nt == _N_TILES - 1)
    def _phase_a():
        mt = jnp.swapaxes(mt_ref[...], 0, 1)
        keys = [mt[16 * w:16 * w + 16] for w in range(32)]
        pay = [jax.lax.broadcasted_iota(jnp.int32, (16, NUM_GROUP), 0) + 16 * w
               for w in range(32)]
        keys, pay = _top32_tournament(keys, pay)
        cid_ref[0] = jnp.concatenate(pay, axis=0)


def _k1_pallas(points8, centers8):
    return pl.pallas_call(
        _k1_kernel,
        grid=(_B, _N_TILES),
        in_specs=[
            pl.BlockSpec((1, 8, _NT), lambda b, n: (b, 0, n)),
            pl.BlockSpec((1, NUM_GROUP, 8), lambda b, n: (b, 0, 0)),
        ],
        out_specs=[
            pl.BlockSpec((1, NUM_GROUP, _NT), lambda b, n: (b, 0, n)),
            pl.BlockSpec((1, 32, NUM_GROUP), lambda b, n: (b, 0, 0)),
        ],
        out_shape=[
            jax.ShapeDtypeStruct((_B, NUM_GROUP, _N_PTS), jnp.float32),
            jax.ShapeDtypeStruct((_B, 32, NUM_GROUP), jnp.int32),
        ],
        scratch_shapes=[pltpu.VMEM((NUM_GROUP, _N_CHUNK), jnp.float32)],
    )(points8, centers8)


def _k3_kernel(t_ref, p0_ref, out_ref):
    T = t_ref[0]
    P0 = p0_ref[0]
    keys = [T[32 * w:32 * w + 32] for w in range(32)]
    pay = [P0[32 * w:32 * w + 32] for w in range(32)]
    keys, pay = _top32_tournament(keys, pay)
    out_ref[0] = jnp.concatenate(pay, axis=0)


def _k3_pallas(T, P0):
    return pl.pallas_call(
        _k3_kernel,
        grid=(_B,),
        in_specs=[
            pl.BlockSpec((1, 1024, NUM_GROUP), lambda b: (b, 0, 0)),
            pl.BlockSpec((1, 1024, NUM_GROUP), lambda b: (b, 0, 0)),
        ],
        out_specs=pl.BlockSpec((1, 32, NUM_GROUP), lambda b: (b, 0, 0)),
        out_shape=jax.ShapeDtypeStruct((_B, 32, NUM_GROUP), jnp.int32),
    )(T, P0)


def kernel(data):
    batch_size, num_points, C = data.shape
    x = data[:, :, 0].reshape(batch_size, _N_SIDE, _N_SIDE)
    y = data[:, :, 1].reshape(batch_size, _N_SIDE, _N_SIDE)
    z = data[:, :, 2].reshape(batch_size, _N_SIDE, _N_SIDE)
    cx, cy, cz = _fps_pallas(x, y, z)
    center = jnp.stack([cx[:, :, 0], cy[:, :, 0], cz[:, :, 0]], axis=-1)

    points8 = jnp.concatenate(
        [jnp.swapaxes(data, 1, 2),
         jnp.zeros((batch_size, 5, num_points), jnp.float32)], axis=1)
    centers8 = jnp.concatenate(
        [center, jnp.zeros((batch_size, NUM_GROUP, 5), jnp.float32)], axis=2)

    D, cid = _k1_pallas(points8, centers8)

    cidx = jnp.swapaxes(cid, 1, 2)
    nidx = (cidx[..., None] * 32
            + jnp.arange(32, dtype=jnp.int32)[None, None, None, :])
    nidx = nidx.reshape(batch_size, NUM_GROUP, 1024)
    T0 = jnp.take_along_axis(D, nidx, axis=2)
    T = jnp.swapaxes(T0, 1, 2)
    P0 = (cid[:, :, None, :] * 32
          + jnp.arange(32, dtype=jnp.int32)[None, None, :, None])
    P0 = P0.reshape(batch_size, 1024, NUM_GROUP)

    nbr = _k3_pallas(T, P0)
    idx = jnp.swapaxes(nbr, 1, 2)

    idx_base = jnp.arange(batch_size).reshape(-1, 1, 1) * num_points
    fidx = (idx + idx_base).reshape(-1)
    neighborhood = data.reshape(batch_size * num_points, 3)[fidx, :]
    neighborhood = neighborhood.reshape(batch_size, NUM_GROUP, GROUP_SIZE, 3)
    neighborhood = neighborhood - center[:, :, None, :]
    return (neighborhood, center)


# speedup vs baseline: 1.5853x; 1.5853x over previous
import functools

import jax
import jax.numpy as jnp
from jax.experimental import pallas as pl
from jax.experimental.pallas import tpu as pltpu

NUM_GROUP = 256
GROUP_SIZE = 32
_N_SIDE = 128
_N_PTS = 16384
_B = 8
_NT = 4096
_N_TILES = _N_PTS // _NT
_N_CHUNK = 512
_CPT = _NT // 32


def _oems_pairs(n):
    pairs = []
    p = 1
    while p < n:
        k = p
        while k >= 1:
            for j in range(k % p, n - k, 2 * k):
                for i in range(0, k):
                    if (i + j) // (2 * p) == (i + j + k) // (2 * p):
                        pairs.append((i + j, i + j + k))
            k //= 2
        p *= 2
    return pairs


def _bitonic_merge_pairs(n):
    pairs = []
    d = n // 2
    while d >= 1:
        for i in range(n):
            if (i & d) == 0 and i + d < n:
                pairs.append((i, i + d))
        d //= 2
    return pairs


_SORT32 = _oems_pairs(32)
_MERGE32 = _bitonic_merge_pairs(32)


def _cmp_lex(keys, pay, i, j):
    ka, kb = keys[i], keys[j]
    pa, pb = pay[i], pay[j]
    pred = (ka < kb) | ((ka == kb) & (pa < pb))
    keys[i] = jnp.where(pred, ka, kb)
    keys[j] = jnp.where(pred, kb, ka)
    pay[i] = jnp.where(pred, pa, pb)
    pay[j] = jnp.where(pred, pb, pa)


def _top32_tournament(keys, pay):
    for i, j in _SORT32:
        _cmp_lex(keys, pay, i, j)
    L = keys[0].shape[0]
    while L > 1:
        h = L // 2
        ka = [keys[w][:h] for w in range(32)]
        kb = [keys[31 - w][h:] for w in range(32)]
        pa = [pay[w][:h] for w in range(32)]
        pb = [pay[31 - w][h:] for w in range(32)]
        for w in range(32):
            pred = (ka[w] < kb[w]) | ((ka[w] == kb[w]) & (pa[w] < pb[w]))
            keys[w] = jnp.where(pred, ka[w], kb[w])
            pay[w] = jnp.where(pred, pa[w], pb[w])
        for i, j in _MERGE32:
            _cmp_lex(keys, pay, i, j)
        L = h
    return keys, pay


def _fps_kernel(x_ref, y_ref, z_ref, cx_ref, cy_ref, cz_ref, dists_ref):
    X = x_ref[...]
    Y = y_ref[...]
    Z = z_ref[...]
    rows = jax.lax.broadcasted_iota(jnp.int32, (_N_SIDE, _N_SIDE), 0)
    cols = jax.lax.broadcasted_iota(jnp.int32, (_N_SIDE, _N_SIDE), 1)
    iota_flat = (rows * _N_SIDE + cols)[None, :, :]
    lane_iota = jax.lax.broadcasted_iota(jnp.int32, (1, _N_SIDE), 1)
    dists_ref[...] = jnp.full((_B, _N_SIDE, _N_SIDE), 1e10, dtype=jnp.float32)

    def body(i, last):
        pxs, pys, pzs = [], [], []
        for b in range(_B):
            r = last[b] // _N_SIDE
            c = last[b] % _N_SIDE
            cmask = lane_iota == c
            pxs.append(jnp.sum(jnp.where(cmask, x_ref[b, pl.ds(r, 1), :], 0.0)))
            pys.append(jnp.sum(jnp.where(cmask, y_ref[b, pl.ds(r, 1), :], 0.0)))
            pzs.append(jnp.sum(jnp.where(cmask, z_ref[b, pl.ds(r, 1), :], 0.0)))
        px = jnp.stack(pxs).reshape(_B, 1, 1)
        py = jnp.stack(pys).reshape(_B, 1, 1)
        pz = jnp.stack(pzs).reshape(_B, 1, 1)
        cx_ref[:, pl.ds(i, 1), :] = jnp.broadcast_to(px, (_B, 1, _N_SIDE))
        cy_ref[:, pl.ds(i, 1), :] = jnp.broadcast_to(py, (_B, 1, _N_SIDE))
        cz_ref[:, pl.ds(i, 1), :] = jnp.broadcast_to(pz, (_B, 1, _N_SIDE))
        dx = X - px
        dy = Y - py
        dz = Z - pz
        d = dx * dx + dy * dy + dz * dz
        dists = jnp.minimum(dists_ref[...], d)
        dists_ref[...] = dists
        m = jnp.max(dists, axis=(1, 2), keepdims=True)
        amask = dists == m
        nxt = jnp.min(jnp.where(amask, iota_flat, jnp.int32(2**30)), axis=(1, 2))
        return nxt
    jax.lax.fori_loop(0, NUM_GROUP, body, jnp.zeros((_B,), jnp.int32))


def _fps_pallas(x, y, z):
    out_shape = jax.ShapeDtypeStruct((_B, NUM_GROUP, _N_SIDE), jnp.float32)
    return pl.pallas_call(
        _fps_kernel,
        out_shape=[out_shape, out_shape, out_shape],
        scratch_shapes=[pltpu.VMEM((_B, _N_SIDE, _N_SIDE), jnp.float32)],
    )(x, y, z)


def _k1_kernel(p8_ref, c8_ref, d_ref, cid_ref, mt_ref):
    nt = pl.program_id(1)
    P = p8_ref[0]
    C = c8_ref[0]
    qq = (C[:, 0:1] * C[:, 0:1] + C[:, 1:2] * C[:, 1:2]) + C[:, 2:3] * C[:, 2:3]
    rr = (P[0:1, :] * P[0:1, :] + P[1:2, :] * P[1:2, :]) + P[2:3, :] * P[2:3, :]
    dot = jax.lax.dot_general(C, P, (((1,), (0,)), ((), ())),
                              preferred_element_type=jnp.float32)
    Dt = (qq + rr) - 2.0 * dot
    d_ref[0] = Dt

    m = jnp.min(Dt.reshape(NUM_GROUP, _CPT, 32), axis=2)
    mt_ref[:, pl.ds(nt * _CPT, _CPT)] = m

    @pl.when(nt == _N_TILES - 1)
    def _phase_a():
        mt = jnp.swapaxes(mt_ref[...], 0, 1)
        keys = [mt[16 * w:16 * w + 16] for w in range(32)]
        pay = [jax.lax.broadcasted_iota(jnp.int32, (16, NUM_GROUP), 0) + 16 * w
               for w in range(32)]
        keys, pay = _top32_tournament(keys, pay)
        cid_ref[0] = jnp.concatenate(pay, axis=0)


def _k1_pallas(points8, centers8):
    return pl.pallas_call(
        _k1_kernel,
        grid=(_B, _N_TILES),
        in_specs=[
            pl.BlockSpec((1, 8, _NT), lambda b, n: (b, 0, n)),
            pl.BlockSpec((1, NUM_GROUP, 8), lambda b, n: (b, 0, 0)),
        ],
        out_specs=[
            pl.BlockSpec((1, NUM_GROUP, _NT), lambda b, n: (b, 0, n)),
            pl.BlockSpec((1, 32, NUM_GROUP), lambda b, n: (b, 0, 0)),
        ],
        out_shape=[
            jax.ShapeDtypeStruct((_B, NUM_GROUP, _N_PTS), jnp.float32),
            jax.ShapeDtypeStruct((_B, 32, NUM_GROUP), jnp.int32),
        ],
        scratch_shapes=[pltpu.VMEM((NUM_GROUP, _N_CHUNK), jnp.float32)],
    )(points8, centers8)


def _k3_kernel(t_ref, p0_ref, out_ref):
    T = t_ref[0]
    P0 = p0_ref[0]
    keys = [T[32 * w:32 * w + 32] for w in range(32)]
    pay = [P0[32 * w:32 * w + 32] for w in range(32)]
    keys, pay = _top32_tournament(keys, pay)
    out_ref[0] = jnp.concatenate(pay, axis=0)


def _k3_pallas(T, P0):
    return pl.pallas_call(
        _k3_kernel,
        grid=(_B,),
        in_specs=[
            pl.BlockSpec((1, 1024, NUM_GROUP), lambda b: (b, 0, 0)),
            pl.BlockSpec((1, 1024, NUM_GROUP), lambda b: (b, 0, 0)),
        ],
        out_specs=pl.BlockSpec((1, 32, NUM_GROUP), lambda b: (b, 0, 0)),
        out_shape=jax.ShapeDtypeStruct((_B, 32, NUM_GROUP), jnp.int32),
    )(T, P0)


def kernel(data):
    batch_size, num_points, C = data.shape
    x = data[:, :, 0].reshape(batch_size, _N_SIDE, _N_SIDE)
    y = data[:, :, 1].reshape(batch_size, _N_SIDE, _N_SIDE)
    z = data[:, :, 2].reshape(batch_size, _N_SIDE, _N_SIDE)
    cx, cy, cz = _fps_pallas(x, y, z)
    center = jnp.stack([cx[:, :, 0], cy[:, :, 0], cz[:, :, 0]], axis=-1)

    points8 = jnp.concatenate(
        [jnp.swapaxes(data, 1, 2),
         jnp.zeros((batch_size, 5, num_points), jnp.float32)], axis=1)
    centers8 = jnp.concatenate(
        [center, jnp.zeros((batch_size, NUM_GROUP, 5), jnp.float32)], axis=2)

    D, cid = _k1_pallas(points8, centers8)

    cidx = jnp.swapaxes(cid, 1, 2)
    nidx = (cidx[..., None] * 32
            + jnp.arange(32, dtype=jnp.int32)[None, None, None, :])
    nidx = nidx.reshape(batch_size, NUM_GROUP, 1024)
    T0 = jnp.take_along_axis(D, nidx, axis=2)
    T = jnp.swapaxes(T0, 1, 2)
    P0 = (cid[:, :, None, :] * 32
          + jnp.arange(32, dtype=jnp.int32)[None, None, :, None])
    P0 = P0.reshape(batch_size, 1024, NUM_GROUP)

    nbr = _k3_pallas(T, P0)
    idx = jnp.swapaxes(nbr, 1, 2)

    idx_base = jnp.arange(batch_size).reshape(-1, 1, 1) * num_points
    fidx = (idx + idx_base).reshape(-1)
    neighborhood = data.reshape(batch_size * num_points, 3)[fidx, :]
    neighborhood = neighborhood.reshape(batch_size, NUM_GROUP, GROUP_SIZE, 3)
    neighborhood = neighborhood - center[:, :, None, :]
    return (neighborhood, center)


# speedup vs baseline: 2.1497x; 1.3561x over previous
import functools

import jax
import jax.numpy as jnp
from jax.experimental import pallas as pl
from jax.experimental.pallas import tpu as pltpu

NUM_GROUP = 256
GROUP_SIZE = 32
_N_SIDE = 128
_N_PTS = 16384
_B = 8
_NT = 4096
_N_TILES = _N_PTS // _NT
_N_CHUNK = 512
_CPT = _NT // 32


def _oems_pairs(n):
    pairs = []
    p = 1
    while p < n:
        k = p
        while k >= 1:
            for j in range(k % p, n - k, 2 * k):
                for i in range(0, k):
                    if (i + j) // (2 * p) == (i + j + k) // (2 * p):
                        pairs.append((i + j, i + j + k))
            k //= 2
        p *= 2
    return pairs


def _bitonic_merge_pairs(n):
    pairs = []
    d = n // 2
    while d >= 1:
        for i in range(n):
            if (i & d) == 0 and i + d < n:
                pairs.append((i, i + d))
        d //= 2
    return pairs


_SORT32 = _oems_pairs(32)
_MERGE32 = _bitonic_merge_pairs(32)


def _cmp_lex(keys, pay, i, j):
    ka, kb = keys[i], keys[j]
    pa, pb = pay[i], pay[j]
    pred = (ka < kb) | ((ka == kb) & (pa < pb))
    keys[i] = jnp.where(pred, ka, kb)
    keys[j] = jnp.where(pred, kb, ka)
    pay[i] = jnp.where(pred, pa, pb)
    pay[j] = jnp.where(pred, pb, pa)


def _top32_tournament(keys, pay):
    for i, j in _SORT32:
        _cmp_lex(keys, pay, i, j)
    L = keys[0].shape[0]
    while L > 1:
        h = L // 2
        ka = [keys[w][:h] for w in range(32)]
        kb = [keys[31 - w][h:] for w in range(32)]
        pa = [pay[w][:h] for w in range(32)]
        pb = [pay[31 - w][h:] for w in range(32)]
        for w in range(32):
            pred = (ka[w] < kb[w]) | ((ka[w] == kb[w]) & (pa[w] < pb[w]))
            keys[w] = jnp.where(pred, ka[w], kb[w])
            pay[w] = jnp.where(pred, pa[w], pb[w])
        for i, j in _MERGE32:
            _cmp_lex(keys, pay, i, j)
        L = h
    return keys, pay


def _fps_kernel(x_ref, y_ref, z_ref, cx_ref, cy_ref, cz_ref, dists_ref):
    X = x_ref[...]
    Y = y_ref[...]
    Z = z_ref[...]
    rows = jax.lax.broadcasted_iota(jnp.int32, (_N_SIDE, _N_SIDE), 0)
    cols = jax.lax.broadcasted_iota(jnp.int32, (_N_SIDE, _N_SIDE), 1)
    iota_flat = (rows * _N_SIDE + cols)[None, :, :]
    dists_ref[...] = jnp.full((_B, _N_SIDE, _N_SIDE), 1e10, dtype=jnp.float32)

    def body(i, last):
        lmask = iota_flat == last.reshape(_B, 1, 1)
        px = jnp.sum(jnp.where(lmask, X, 0.0), axis=(1, 2), keepdims=True)
        py = jnp.sum(jnp.where(lmask, Y, 0.0), axis=(1, 2), keepdims=True)
        pz = jnp.sum(jnp.where(lmask, Z, 0.0), axis=(1, 2), keepdims=True)
        cx_ref[:, pl.ds(i, 1), :] = jnp.broadcast_to(px, (_B, 1, _N_SIDE))
        cy_ref[:, pl.ds(i, 1), :] = jnp.broadcast_to(py, (_B, 1, _N_SIDE))
        cz_ref[:, pl.ds(i, 1), :] = jnp.broadcast_to(pz, (_B, 1, _N_SIDE))
        dx = X - px
        dy = Y - py
        dz = Z - pz
        d = dx * dx + dy * dy + dz * dz
        dists = jnp.minimum(dists_ref[...], d)
        dists_ref[...] = dists
        m = jnp.max(dists, axis=(1, 2), keepdims=True)
        amask = dists == m
        nxt = jnp.min(jnp.where(amask, iota_flat, jnp.int32(2**30)), axis=(1, 2))
        return nxt
    jax.lax.fori_loop(0, NUM_GROUP, body, jnp.zeros((_B,), jnp.int32))


def _fps_pallas(x, y, z):
    out_shape = jax.ShapeDtypeStruct((_B, NUM_GROUP, _N_SIDE), jnp.float32)
    return pl.pallas_call(
        _fps_kernel,
        out_shape=[out_shape, out_shape, out_shape],
        scratch_shapes=[pltpu.VMEM((_B, _N_SIDE, _N_SIDE), jnp.float32)],
    )(x, y, z)


def _k1_kernel(p8_ref, c8_ref, d_ref, cid_ref, mt_ref):
    nt = pl.program_id(1)
    P = p8_ref[0]
    C = c8_ref[0]
    qq = (C[:, 0:1] * C[:, 0:1] + C[:, 1:2] * C[:, 1:2]) + C[:, 2:3] * C[:, 2:3]
    rr = (P[0:1, :] * P[0:1, :] + P[1:2, :] * P[1:2, :]) + P[2:3, :] * P[2:3, :]
    dot = jax.lax.dot_general(C, P, (((1,), (0,)), ((), ())),
                              preferred_element_type=jnp.float32)
    Dt = (qq + rr) - 2.0 * dot
    d_ref[0] = Dt

    m = jnp.min(Dt.reshape(NUM_GROUP, _CPT, 32), axis=2)
    mt_ref[:, pl.ds(nt * _CPT, _CPT)] = m

    @pl.when(nt == _N_TILES - 1)
    def _phase_a():
        mt = jnp.swapaxes(mt_ref[...], 0, 1)
        keys = [mt[16 * w:16 * w + 16] for w in range(32)]
        pay = [jax.lax.broadcasted_iota(jnp.int32, (16, NUM_GROUP), 0) + 16 * w
               for w in range(32)]
        keys, pay = _top32_tournament(keys, pay)
        cid_ref[0] = jnp.concatenate(pay, axis=0)


def _k1_pallas(points8, centers8):
    return pl.pallas_call(
        _k1_kernel,
        grid=(_B, _N_TILES),
        in_specs=[
            pl.BlockSpec((1, 8, _NT), lambda b, n: (b, 0, n)),
            pl.BlockSpec((1, NUM_GROUP, 8), lambda b, n: (b, 0, 0)),
        ],
        out_specs=[
            pl.BlockSpec((1, NUM_GROUP, _NT), lambda b, n: (b, 0, n)),
            pl.BlockSpec((1, 32, NUM_GROUP), lambda b, n: (b, 0, 0)),
        ],
        out_shape=[
            jax.ShapeDtypeStruct((_B, NUM_GROUP, _N_PTS), jnp.float32),
            jax.ShapeDtypeStruct((_B, 32, NUM_GROUP), jnp.int32),
        ],
        scratch_shapes=[pltpu.VMEM((NUM_GROUP, _N_CHUNK), jnp.float32)],
    )(points8, centers8)


def _k3_kernel(t_ref, p0_ref, out_ref):
    T = t_ref[0]
    P0 = p0_ref[0]
    keys = [T[32 * w:32 * w + 32] for w in range(32)]
    pay = [P0[32 * w:32 * w + 32] for w in range(32)]
    keys, pay = _top32_tournament(keys, pay)
    out_ref[0] = jnp.concatenate(pay, axis=0)


def _k3_pallas(T, P0):
    return pl.pallas_call(
        _k3_kernel,
        grid=(_B,),
        in_specs=[
            pl.BlockSpec((1, 1024, NUM_GROUP), lambda b: (b, 0, 0)),
            pl.BlockSpec((1, 1024, NUM_GROUP), lambda b: (b, 0, 0)),
        ],
        out_specs=pl.BlockSpec((1, 32, NUM_GROUP), lambda b: (b, 0, 0)),
        out_shape=jax.ShapeDtypeStruct((_B, 32, NUM_GROUP), jnp.int32),
    )(T, P0)


def kernel(data):
    batch_size, num_points, C = data.shape
    x = data[:, :, 0].reshape(batch_size, _N_SIDE, _N_SIDE)
    y = data[:, :, 1].reshape(batch_size, _N_SIDE, _N_SIDE)
    z = data[:, :, 2].reshape(batch_size, _N_SIDE, _N_SIDE)
    cx, cy, cz = _fps_pallas(x, y, z)
    center = jnp.stack([cx[:, :, 0], cy[:, :, 0], cz[:, :, 0]], axis=-1)

    points8 = jnp.concatenate(
        [jnp.swapaxes(data, 1, 2),
         jnp.zeros((batch_size, 5, num_points), jnp.float32)], axis=1)
    centers8 = jnp.concatenate(
        [center, jnp.zeros((batch_size, NUM_GROUP, 5), jnp.float32)], axis=2)

    D, cid = _k1_pallas(points8, centers8)

    cidx = jnp.swapaxes(cid, 1, 2)
    nidx = (cidx[..., None] * 32
            + jnp.arange(32, dtype=jnp.int32)[None, None, None, :])
    nidx = nidx.reshape(batch_size, NUM_GROUP, 1024)
    T0 = jnp.take_along_axis(D, nidx, axis=2)
    T = jnp.swapaxes(T0, 1, 2)
    P0 = (cid[:, :, None, :] * 32
          + jnp.arange(32, dtype=jnp.int32)[None, None, :, None])
    P0 = P0.reshape(batch_size, 1024, NUM_GROUP)

    nbr = _k3_pallas(T, P0)
    idx = jnp.swapaxes(nbr, 1, 2)

    idx_base = jnp.arange(batch_size).reshape(-1, 1, 1) * num_points
    fidx = (idx + idx_base).reshape(-1)
    neighborhood = data.reshape(batch_size * num_points, 3)[fidx, :]
    neighborhood = neighborhood.reshape(batch_size, NUM_GROUP, GROUP_SIZE, 3)
    neighborhood = neighborhood - center[:, :, None, :]
    return (neighborhood, center)
